# baseline (speedup 1.0000x reference)
import functools

import jax
import jax.numpy as jnp
from jax import lax
from jax.experimental import pallas as pl
from jax.experimental.pallas import tpu as pltpu
from jax.experimental.pallas import tpu_sc as plsc

_NC = 2
_NS = 16
_NW = _NC * _NS
_L = 16
_NBUF = 4


def _project_table(emb, W1):
    V, E = emb.shape
    H = W1.shape[1]
    VB = 10000

    def body(emb_ref, w1_ref, out_ref):
        r = jnp.dot(emb_ref[...], w1_ref[...],
                    preferred_element_type=jnp.float32)
        out_ref[:, 0:H] = r
        out_ref[:, H:2 * H] = r

    return pl.pallas_call(
        body,
        grid=(V // VB,),
        in_specs=[pl.BlockSpec((VB, E), lambda i: (i, 0)),
                  pl.BlockSpec((E, H), lambda i: (0, 0))],
        out_specs=pl.BlockSpec((VB, 2 * H), lambda i: (i, 0)),
        out_shape=jax.ShapeDtypeStruct((V, 2 * H), jnp.float32),
    )(emb, W1).reshape(2 * V, H)


def _transpose_idx(x, B, S, bpw):
    def body(x_ref, out_ref):
        out_ref[...] = jnp.transpose(x_ref[...]) * 2

    return pl.pallas_call(
        body,
        grid=(B // bpw,),
        in_specs=[pl.BlockSpec((bpw, S), lambda i: (i, 0))],
        out_specs=pl.BlockSpec((S, bpw), lambda i: (i, 0)),
        out_shape=jax.ShapeDtypeStruct(((B // bpw) * S, bpw), jnp.int32),
    )(x)


def _sc_pool(table, xsc, B, S, H, bpw):
    mesh = plsc.VectorSubcoreMesh(core_axis_name="c", subcore_axis_name="s")

    @functools.partial(
        pl.kernel,
        out_type=jax.ShapeDtypeStruct((B, H), jnp.float32),
        mesh=mesh,
        compiler_params=pltpu.CompilerParams(use_tc_tiling_on_sc=False,
                                             needs_layout_passes=False),
        scratch_types=[
            pltpu.VMEM((S, bpw), jnp.int32),
            pltpu.VMEM((_NBUF, bpw, H), jnp.float32),
            pltpu.VMEM((bpw, H), jnp.float32),
            [pltpu.SemaphoreType.DMA] * _NBUF,
        ],
    )
    def k(table_hbm, xsc_hbm, out_hbm, idx_v, bufs, acc_v, sems):
        w = lax.axis_index("s") * _NC + lax.axis_index("c")
        pltpu.sync_copy(xsc_hbm.at[pl.ds(w * S, S)], idx_v)

        zero = jnp.zeros((_L,), jnp.float32)

        def zrow(i, carry):
            for j in range(H // _L):
                acc_v[i, pl.ds(j * _L, _L)] = zero
            return carry

        lax.fori_loop(0, bpw, zrow, None, unroll=8)

        def accum(buf):
            def row(i, carry):
                for j in range(H // _L):
                    sl = (i, pl.ds(j * _L, _L))
                    plsc.addupdate(acc_v.at[sl], buf[sl])
                return carry
            lax.fori_loop(0, bpw, row, None, unroll=8)

        for b in range(_NBUF):
            pltpu.async_copy(table_hbm.at[idx_v.at[b]], bufs.at[b], sems[b])

        def step(it, carry):
            s = it * _NBUF
            for b in range(_NBUF):
                pltpu.make_async_copy(
                    table_hbm.at[idx_v.at[s + b]], bufs.at[b], sems[b]
                ).wait()
                accum(bufs.at[b])

                @pl.when(s + b + _NBUF < S)
                def _():
                    pltpu.async_copy(
                        table_hbm.at[idx_v.at[s + b + _NBUF]],
                        bufs.at[b], sems[b])
            return carry

        lax.fori_loop(0, S // _NBUF, step, None)
        pltpu.sync_copy(acc_v, out_hbm.at[pl.ds(w * bpw, bpw)])

    return k(table, xsc)


def _mlp(pooled_sum, b1, W2, b2, S):
    B, H = pooled_sum.shape
    C = W2.shape[1]
    BB = 512
    inv_s = 1.0 / S

    def body(ps_ref, b1_ref, w2_ref, b2_ref, out_ref):
        h = ps_ref[...] * inv_s + b1_ref[...]
        h = jnp.maximum(h, 0.0)
        out_ref[...] = jnp.dot(h, w2_ref[...],
                               preferred_element_type=jnp.float32) + b2_ref[...]

    return pl.pallas_call(
        body,
        grid=(B // BB,),
        in_specs=[pl.BlockSpec((BB, H), lambda i: (i, 0)),
                  pl.BlockSpec((1, H), lambda i: (0, 0)),
                  pl.BlockSpec((H, C), lambda i: (0, 0)),
                  pl.BlockSpec((1, C), lambda i: (0, 0))],
        out_specs=pl.BlockSpec((BB, C), lambda i: (i, 0)),
        out_shape=jax.ShapeDtypeStruct((B, C), jnp.float32),
    )(pooled_sum, b1.reshape(1, H), W2, b2.reshape(1, C))


def kernel(x, emb, W1, b1, W2, b2):
    B, S = x.shape
    H = W1.shape[1]
    bpw = B // _NW
    table = _project_table(emb, W1)
    xsc = _transpose_idx(x, B, S, bpw)
    pooled_sum = _sc_pool(table, xsc, B, S, H, bpw)
    return _mlp(pooled_sum, b1, W2, b2, S)


# speedup vs baseline: 1.0181x; 1.0181x over previous
import functools

import jax
import jax.numpy as jnp
from jax import lax
from jax.experimental import pallas as pl
from jax.experimental.pallas import tpu as pltpu
from jax.experimental.pallas import tpu_sc as plsc

_NC = 2
_NS = 16
_NW = _NC * _NS
_L = 16
_NBUF = 4


def _project_table(emb, W1):
    V, E = emb.shape
    H = W1.shape[1]
    VB = 10000

    def body(emb_ref, w1_ref, out_ref):
        r = jnp.dot(emb_ref[...], w1_ref[...],
                    preferred_element_type=jnp.float32)
        out_ref[:, 0:H] = r
        out_ref[:, H:2 * H] = r

    return pl.pallas_call(
        body,
        grid=(V // VB,),
        in_specs=[pl.BlockSpec((VB, E), lambda i: (i, 0)),
                  pl.BlockSpec((E, H), lambda i: (0, 0))],
        out_specs=pl.BlockSpec((VB, 2 * H), lambda i: (i, 0)),
        out_shape=jax.ShapeDtypeStruct((V, 2 * H), jnp.float32),
    )(emb, W1).reshape(2 * V, H)


def _sc_pool(table, x, B, S, H, bpw):
    mesh = plsc.VectorSubcoreMesh(core_axis_name="c", subcore_axis_name="s")

    @functools.partial(
        pl.kernel,
        out_type=jax.ShapeDtypeStruct((B, H), jnp.float32),
        mesh=mesh,
        compiler_params=pltpu.CompilerParams(use_tc_tiling_on_sc=False,
                                             needs_layout_passes=False),
        scratch_types=[
            pltpu.VMEM((bpw, S), jnp.int32),
            pltpu.VMEM((_NBUF, bpw), jnp.int32),
            pltpu.VMEM((_NBUF, bpw, H), jnp.float32),
            pltpu.VMEM((bpw, H), jnp.float32),
            [pltpu.SemaphoreType.DMA] * _NBUF,
        ],
    )
    def k(table_hbm, x_hbm, out_hbm, xraw_v, idx_v, bufs, acc_v, sems):
        w = lax.axis_index("s") * _NC + lax.axis_index("c")
        pltpu.sync_copy(x_hbm.at[pl.ds(w * bpw, bpw)], xraw_v)

        lanes = lax.iota(jnp.int32, _L)

        def trow_one(s, b):
            for kk in range(bpw // _L):
                rows = lanes + (kk * _L)
                cols = jnp.full((_L,), 0, jnp.int32) + s
                v = plsc.load_gather(xraw_v, [rows, cols])
                idx_v[b, pl.ds(kk * _L, _L)] = v + v

        zero = jnp.zeros((_L,), jnp.float32)

        def zrow(i, carry):
            for j in range(H // _L):
                acc_v[i, pl.ds(j * _L, _L)] = zero
            return carry

        lax.fori_loop(0, bpw, zrow, None, unroll=8)

        def accum(buf):
            def row(i, carry):
                for j in range(H // _L):
                    sl = (i, pl.ds(j * _L, _L))
                    plsc.addupdate(acc_v.at[sl], buf[sl])
                return carry
            lax.fori_loop(0, bpw, row, None, unroll=8)

        for b in range(_NBUF):
            trow_one(b, b)
            pltpu.async_copy(table_hbm.at[idx_v.at[b]], bufs.at[b], sems[b])

        def step(it, carry):
            s = it * _NBUF
            for b in range(_NBUF):
                pltpu.make_async_copy(
                    table_hbm.at[idx_v.at[b]], bufs.at[b], sems[b]
                ).wait()
                accum(bufs.at[b])

                @pl.when(s + b + _NBUF < S)
                def _():
                    trow_one(s + b + _NBUF, b)
                    pltpu.async_copy(
                        table_hbm.at[idx_v.at[b]],
                        bufs.at[b], sems[b])
            return carry

        lax.fori_loop(0, S // _NBUF, step, None)
        pltpu.sync_copy(acc_v, out_hbm.at[pl.ds(w * bpw, bpw)])

    return k(table, x)


def _mlp(pooled_sum, b1, W2, b2, S):
    B, H = pooled_sum.shape
    C = W2.shape[1]
    BB = 512
    inv_s = 1.0 / S

    def body(ps_ref, b1_ref, w2_ref, b2_ref, out_ref):
        h = ps_ref[...] * inv_s + b1_ref[...]
        h = jnp.maximum(h, 0.0)
        out_ref[...] = jnp.dot(h, w2_ref[...],
                               preferred_element_type=jnp.float32) + b2_ref[...]

    return pl.pallas_call(
        body,
        grid=(B // BB,),
        in_specs=[pl.BlockSpec((BB, H), lambda i: (i, 0)),
                  pl.BlockSpec((1, H), lambda i: (0, 0)),
                  pl.BlockSpec((H, C), lambda i: (0, 0)),
                  pl.BlockSpec((1, C), lambda i: (0, 0))],
        out_specs=pl.BlockSpec((BB, C), lambda i: (i, 0)),
        out_shape=jax.ShapeDtypeStruct((B, C), jnp.float32),
    )(pooled_sum, b1.reshape(1, H), W2, b2.reshape(1, C))


def kernel(x, emb, W1, b1, W2, b2):
    B, S = x.shape
    H = W1.shape[1]
    bpw = B // _NW
    table = _project_table(emb, W1)
    pooled_sum = _sc_pool(table, x, B, S, H, bpw)
    return _mlp(pooled_sum, b1, W2, b2, S)


# speedup vs baseline: 1.3529x; 1.3289x over previous
import functools

import jax
import jax.numpy as jnp
from jax import lax
from jax.experimental import pallas as pl
from jax.experimental.pallas import tpu as pltpu
from jax.experimental.pallas import tpu_sc as plsc

_NC = 2
_NS = 16
_NW = _NC * _NS
_L = 16
_NBUF = 4


def _project_table(emb, W1):
    V, E = emb.shape
    H = W1.shape[1]
    VB = 10000

    def body(emb_ref, w1_ref, out_ref):
        r = jnp.dot(emb_ref[...], w1_ref[...],
                    preferred_element_type=jnp.float32)
        out_ref[:, 0:H] = r
        out_ref[:, H:2 * H] = r

    return pl.pallas_call(
        body,
        grid=(V // VB,),
        in_specs=[pl.BlockSpec((VB, E), lambda i: (i, 0)),
                  pl.BlockSpec((E, H), lambda i: (0, 0))],
        out_specs=pl.BlockSpec((VB, 2 * H), lambda i: (i, 0)),
        out_shape=jax.ShapeDtypeStruct((V, 2 * H), jnp.float32),
    )(emb, W1).reshape(2 * V, H)


def _sc_pool(table, x, B, S, H, bpw):
    mesh = plsc.VectorSubcoreMesh(core_axis_name="c", subcore_axis_name="s")

    @functools.partial(
        pl.kernel,
        out_type=jax.ShapeDtypeStruct((B, H), jnp.float32),
        mesh=mesh,
        compiler_params=pltpu.CompilerParams(use_tc_tiling_on_sc=False,
                                             needs_layout_passes=False),
        scratch_types=[
            pltpu.VMEM((bpw, S), jnp.int32),
            pltpu.VMEM((_NBUF, S, H), jnp.float32),
            pltpu.VMEM((bpw, H), jnp.float32),
            [pltpu.SemaphoreType.DMA] * _NBUF,
        ],
    )
    def k(table_hbm, x_hbm, out_hbm, xraw_v, bufs, acc_v, sems):
        w = lax.axis_index("s") * _NC + lax.axis_index("c")
        pltpu.sync_copy(x_hbm.at[pl.ds(w * bpw, bpw)], xraw_v)

        h1 = 104
        h2 = S - h1

        def arm(j, b):
            pltpu.async_copy(
                table_hbm.at[xraw_v.at[j, pl.ds(0, h1)]],
                bufs.at[b, pl.ds(0, h1)], sems[b])
            pltpu.async_copy(
                table_hbm.at[xraw_v.at[j, pl.ds(h1, h2)]],
                bufs.at[b, pl.ds(h1, h2)], sems[b])

        def wait(j, b):
            pltpu.make_async_copy(
                table_hbm.at[xraw_v.at[j, pl.ds(0, h1)]],
                bufs.at[b, pl.ds(0, h1)], sems[b]).wait()
            pltpu.make_async_copy(
                table_hbm.at[xraw_v.at[j, pl.ds(h1, h2)]],
                bufs.at[b, pl.ds(h1, h2)], sems[b]).wait()

        for b in range(_NBUF):
            arm(b, b)

        zero = jnp.zeros((_L,), jnp.float32)

        def group(it, carry):
            for b in range(_NBUF):
                j = it * _NBUF + b
                wait(j, b)
                buf = bufs.at[b]
                accs = (zero,) * (H // _L)

                def srow(s, a):
                    return tuple(
                        a[c] + buf[s, pl.ds(c * _L, _L)]
                        for c in range(H // _L))

                accs = lax.fori_loop(0, S, srow, accs, unroll=8)
                for c in range(H // _L):
                    acc_v[j, pl.ds(c * _L, _L)] = accs[c]

                @pl.when(j + _NBUF < bpw)
                def _():
                    arm(j + _NBUF, b)
            return carry

        lax.fori_loop(0, bpw // _NBUF, group, None)
        pltpu.sync_copy(acc_v, out_hbm.at[pl.ds(w * bpw, bpw)])

    return k(table, x)


def _mlp(pooled_sum, b1, W2, b2, S):
    B, H = pooled_sum.shape
    C = W2.shape[1]
    BB = 512
    inv_s = 1.0 / S

    def body(ps_ref, b1_ref, w2_ref, b2_ref, out_ref):
        h = ps_ref[...] * inv_s + b1_ref[...]
        h = jnp.maximum(h, 0.0)
        out_ref[...] = jnp.dot(h, w2_ref[...],
                               preferred_element_type=jnp.float32) + b2_ref[...]

    return pl.pallas_call(
        body,
        grid=(B // BB,),
        in_specs=[pl.BlockSpec((BB, H), lambda i: (i, 0)),
                  pl.BlockSpec((1, H), lambda i: (0, 0)),
                  pl.BlockSpec((H, C), lambda i: (0, 0)),
                  pl.BlockSpec((1, C), lambda i: (0, 0))],
        out_specs=pl.BlockSpec((BB, C), lambda i: (i, 0)),
        out_shape=jax.ShapeDtypeStruct((B, C), jnp.float32),
    )(pooled_sum, b1.reshape(1, H), W2, b2.reshape(1, C))


def kernel(x, emb, W1, b1, W2, b2):
    B, S = x.shape
    H = W1.shape[1]
    bpw = B // _NW
    table = _project_table(emb, W1)
    pooled_sum = _sc_pool(table, x + x, B, S, H, bpw)
    return _mlp(pooled_sum, b1, W2, b2, S)
